# baseline (speedup 1.0000x reference)
import functools

import jax
import jax.numpy as jnp
from jax import lax
from jax.experimental import pallas as pl
from jax.experimental.pallas import tpu as pltpu
from jax.experimental.pallas import tpu_sc as plsc

VOCAB = 100000
EMBED_DIM = 128
BATCH = 4096
HIST = 50
HIST_PAD = 56
NUM_WORKERS = 32
B_PER_W = BATCH // NUM_WORKERS
NB = 4
NCHUNK = B_PER_W // NB
NBUF = 4
GROUPS = NCHUNK // NBUF


def _make_gather():
    mesh = plsc.VectorSubcoreMesh(core_axis_name="c", subcore_axis_name="s")

    scratch = [pltpu.VMEM((B_PER_W, HIST_PAD), jnp.int32)]
    scratch += [
        pltpu.VMEM((NB, HIST_PAD, EMBED_DIM), jnp.float32) for _ in range(NBUF)
    ]
    scratch += [pltpu.SemaphoreType.DMA for _ in range(2 * NBUF)]

    @functools.partial(
        pl.kernel,
        mesh=mesh,
        out_type=jax.ShapeDtypeStruct((BATCH, HIST, EMBED_DIM), jnp.float32),
        scratch_types=scratch,
    )
    def gather_kernel(idx_hbm, table_hbm, out_hbm, idx_v, *bufs_and_sems):
        bufs = bufs_and_sems[:NBUF]
        gsem = bufs_and_sems[NBUF:2 * NBUF]
        psem = bufs_and_sems[2 * NBUF:]
        wid = lax.axis_index("s") * 2 + lax.axis_index("c")
        wbase = wid * B_PER_W
        pltpu.sync_copy(idx_hbm.at[pl.ds(wbase, B_PER_W)], idx_v)

        def fire_gathers(j, s):
            for i in range(NB):
                pltpu.async_copy(
                    table_hbm.at[idx_v.at[j * NB + i]], bufs[s].at[i], gsem[s]
                )

        def wait_gathers(j, s):
            for i in range(NB):
                pltpu.make_async_copy(
                    table_hbm.at[idx_v.at[j * NB + i]], bufs[s].at[i], gsem[s]
                ).wait()

        def fire_put(j, s):
            pltpu.async_copy(
                bufs[s].at[:, pl.ds(0, HIST), :],
                out_hbm.at[pl.ds(wbase + j * NB, NB)],
                psem[s],
            )

        def wait_put(j, s):
            pltpu.make_async_copy(
                bufs[s].at[:, pl.ds(0, HIST), :],
                out_hbm.at[pl.ds(wbase + j * NB, NB)],
                psem[s],
            ).wait()

        for s in range(NBUF):
            fire_gathers(s, s)

        def body(g, carry):
            for s in range(NBUF):
                j = g * NBUF + s
                wait_gathers(j, s)
                fire_put(j, s)
            for s in range(NBUF):
                j = g * NBUF + s
                wait_put(j, s)
                fire_gathers((g + 1) * NBUF + s, s)
            return carry

        lax.fori_loop(0, GROUPS - 1, body, 0)

        g = GROUPS - 1
        for s in range(NBUF):
            j = g * NBUF + s
            wait_gathers(j, s)
            fire_put(j, s)
        for s in range(NBUF):
            wait_put(g * NBUF + s, s)

    return gather_kernel


_gather = _make_gather()


def kernel(vocab_id_list, table):
    idx = jnp.pad(vocab_id_list, ((0, 0), (0, HIST_PAD - HIST)))
    return _gather(idx, table)


# speedup vs baseline: 7.5214x; 7.5214x over previous
import functools

import jax
import jax.numpy as jnp
from jax import lax
from jax.experimental import pallas as pl
from jax.experimental.pallas import tpu as pltpu
from jax.experimental.pallas import tpu_sc as plsc

VOCAB = 100000
EMBED_DIM = 128
BATCH = 4096
HIST = 50
HIST_PAD = 56
NUM_WORKERS = 32
B_PER_W = BATCH // NUM_WORKERS
NBUF = 8
GROUPS = B_PER_W // NBUF


def _make_gather():
    mesh = plsc.VectorSubcoreMesh(core_axis_name="c", subcore_axis_name="s")

    scratch = [pltpu.VMEM((B_PER_W, HIST_PAD), jnp.int32)]
    scratch += [pltpu.VMEM((HIST, EMBED_DIM), jnp.float32) for _ in range(NBUF)]
    scratch += [pltpu.SemaphoreType.DMA for _ in range(2 * NBUF)]

    @functools.partial(
        pl.kernel,
        mesh=mesh,
        out_type=jax.ShapeDtypeStruct((BATCH, HIST, EMBED_DIM), jnp.float32),
        scratch_types=scratch,
    )
    def gather_kernel(idx_hbm, table_hbm, out_hbm, idx_v, *bufs_and_sems):
        bufs = bufs_and_sems[:NBUF]
        gsem = bufs_and_sems[NBUF:2 * NBUF]
        psem = bufs_and_sems[2 * NBUF:]
        wid = lax.axis_index("s") * 2 + lax.axis_index("c")
        wbase = wid * B_PER_W
        pltpu.sync_copy(idx_hbm.at[pl.ds(wbase, B_PER_W)], idx_v)

        def fire_gather(b, s):
            pltpu.async_copy(
                table_hbm.at[idx_v.at[b, pl.ds(0, HIST)]], bufs[s], gsem[s]
            )

        def wait_gather(b, s):
            pltpu.make_async_copy(
                table_hbm.at[idx_v.at[b, pl.ds(0, HIST)]], bufs[s], gsem[s]
            ).wait()

        def fire_put(b, s):
            pltpu.async_copy(bufs[s], out_hbm.at[wbase + b], psem[s])

        def wait_put(b, s):
            pltpu.make_async_copy(
                bufs[s], out_hbm.at[wbase + b], psem[s]
            ).wait()

        for s in range(NBUF):
            fire_gather(s, s)

        def body(g, carry):
            for s in range(NBUF):
                b = g * NBUF + s
                wait_gather(b, s)
                fire_put(b, s)
            for s in range(NBUF):
                b = g * NBUF + s
                wait_put(b, s)
                fire_gather((g + 1) * NBUF + s, s)
            return carry

        lax.fori_loop(0, GROUPS - 1, body, 0)

        g = GROUPS - 1
        for s in range(NBUF):
            b = g * NBUF + s
            wait_gather(b, s)
            fire_put(b, s)
        for s in range(NBUF):
            wait_put(g * NBUF + s, s)

    return gather_kernel


_gather = _make_gather()


def kernel(vocab_id_list, table):
    idx = jnp.pad(vocab_id_list, ((0, 0), (0, HIST_PAD - HIST)))
    return _gather(idx, table)


# speedup vs baseline: 13.2253x; 1.7583x over previous
import functools

import jax
import jax.numpy as jnp
from jax import lax
from jax.experimental import pallas as pl
from jax.experimental.pallas import tpu as pltpu
from jax.experimental.pallas import tpu_sc as plsc

VOCAB = 100000
EMBED_DIM = 128
BATCH = 4096
HIST = 50
NUM_WORKERS = 32
B_PER_W = BATCH // NUM_WORKERS
NBUF = 5
GROUPS = HIST // NBUF


def _make_gather():
    mesh = plsc.VectorSubcoreMesh(core_axis_name="c", subcore_axis_name="s")

    scratch = [pltpu.VMEM((HIST, B_PER_W), jnp.int32)]
    scratch += [
        pltpu.VMEM((B_PER_W, EMBED_DIM), jnp.float32) for _ in range(NBUF)
    ]
    scratch += [pltpu.SemaphoreType.DMA for _ in range(2 * NBUF)]

    @functools.partial(
        pl.kernel,
        mesh=mesh,
        out_type=jax.ShapeDtypeStruct((HIST, BATCH, EMBED_DIM), jnp.float32),
        scratch_types=scratch,
    )
    def gather_kernel(idx_hbm, table_hbm, out_hbm, idx_v, *bufs_and_sems):
        bufs = bufs_and_sems[:NBUF]
        gsem = bufs_and_sems[NBUF:2 * NBUF]
        psem = bufs_and_sems[2 * NBUF:]
        wid = lax.axis_index("s") * 2 + lax.axis_index("c")
        cbase = wid * B_PER_W
        pltpu.sync_copy(idx_hbm.at[:, pl.ds(cbase, B_PER_W)], idx_v)

        def fire_gather(l, s):
            pltpu.async_copy(table_hbm.at[idx_v.at[l]], bufs[s], gsem[s])

        def wait_gather(l, s):
            pltpu.make_async_copy(
                table_hbm.at[idx_v.at[l]], bufs[s], gsem[s]
            ).wait()

        def fire_put(l, s):
            pltpu.async_copy(
                bufs[s], out_hbm.at[l, pl.ds(cbase, B_PER_W)], psem[s]
            )

        def wait_put(l, s):
            pltpu.make_async_copy(
                bufs[s], out_hbm.at[l, pl.ds(cbase, B_PER_W)], psem[s]
            ).wait()

        for s in range(NBUF):
            fire_gather(s, s)

        def body(g, carry):
            for s in range(NBUF):
                l = g * NBUF + s
                wait_gather(l, s)
                fire_put(l, s)
            for s in range(NBUF):
                l = g * NBUF + s
                wait_put(l, s)
                fire_gather((g + 1) * NBUF + s, s)
            return carry

        lax.fori_loop(0, GROUPS - 1, body, 0)

        g = GROUPS - 1
        for s in range(NBUF):
            l = g * NBUF + s
            wait_gather(l, s)
            fire_put(l, s)
        for s in range(NBUF):
            wait_put(g * NBUF + s, s)

    return gather_kernel


_gather = _make_gather()


def kernel(vocab_id_list, table):
    idx_t = vocab_id_list.T
    out_t = _gather(idx_t, table)
    return out_t.transpose(1, 0, 2)
